# baseline (speedup 1.0000x reference)
import functools

import jax
import jax.numpy as jnp
from jax import lax
from jax.experimental import pallas as pl
from jax.experimental.pallas import tpu as pltpu
from jax.experimental.pallas import tpu_sc as plsc

_N = 10000
_D = 128
_NPAD = 10240
_SHEET = _NPAD // 128
_NC = 2
_NS = 16
_NT = _NC * _NS
_ROWS_PER_TILE = _NPAD // _NS
_RCHUNKS = _ROWS_PER_TILE // 128

_CH_E = 80
_CH_M = 40
_PASS = 40
_SPLIT_E = (160, 0)
_SPLIT_M = (80, 0)


def _mesh():
    return plsc.VectorSubcoreMesh(
        core_axis_name="c", subcore_axis_name="s",
        num_cores=_NC, num_subcores=_NS)


def _sc_hist(dst_e, dst_m):
    chmax = max(_CH_E, _CH_M)

    @functools.partial(
        pl.kernel,
        out_type=[jax.ShapeDtypeStruct((_NT, _SHEET, 128), jnp.float32),
                  jax.ShapeDtypeStruct((_NT, _SHEET, 128), jnp.float32)],
        mesh=_mesh(),
        compiler_params=pltpu.CompilerParams(needs_layout_passes=False),
        name="sc_hist",
        scratch_types=[
            pltpu.VMEM((chmax, 128), jnp.int32),
            pltpu.VMEM((_NPAD,), jnp.float32),
            pltpu.VMEM((_SHEET, 128), jnp.float32),
        ],
    )
    def k(dst_e_hbm, dst_m_hbm, out_e_hbm, out_m_hbm, dstv, histl, hist2d):
        c = lax.axis_index("c")
        s = lax.axis_index("s")
        wid = s * _NC + c
        zero16 = jnp.zeros((16,), jnp.float32)
        ones16 = jnp.ones((16,), jnp.float32)

        def run_graph(dst_hbm, ch, out_hbm):
            def zrow(r, _):
                for g in range(8):
                    histl[pl.ds(r * 128 + g * 16, 16)] = zero16
                return 0
            lax.fori_loop(0, _SHEET, zrow, 0)
            pltpu.sync_copy(dst_hbm.at[pl.ds(wid * ch, ch)],
                            dstv.at[pl.ds(0, ch)])

            def erow(r, _):
                for g in range(8):
                    d = dstv[r, pl.ds(g * 16, 16)]
                    plsc.addupdate_scatter(histl, [d], ones16)
                return 0
            lax.fori_loop(0, ch, erow, 0)

            def prow(r, _):
                for g in range(8):
                    hist2d[r, pl.ds(g * 16, 16)] = (
                        histl[pl.ds(r * 128 + g * 16, 16)])
                return 0
            lax.fori_loop(0, _SHEET, prow, 0)
            pltpu.sync_copy(hist2d, out_hbm.at[wid])

        run_graph(dst_e_hbm, _CH_E, out_e_hbm)
        run_graph(dst_m_hbm, _CH_M, out_m_hbm)

    return k(dst_e, dst_m)


def _sc_scatter(h, src_a, dst_a, split):
    assert split[1] == 0

    @functools.partial(
        pl.kernel,
        out_type=jax.ShapeDtypeStruct((_NPAD, 128), jnp.float32),
        mesh=_mesh(),
        scratch_types=[
            pltpu.VMEM((_PASS, 128), jnp.int32),
            pltpu.VMEM((_PASS, 128), jnp.int32),
            pltpu.VMEM((128, 128), jnp.float32),
            pltpu.VMEM((128, 128), jnp.float32),
            pltpu.VMEM_SHARED((_NPAD, 128), jnp.float32),
            pltpu.SemaphoreType.DMA,
            pltpu.SemaphoreType.DMA,
        ],
    )
    def k(h_hbm, src_hbm, dst_hbm, out_hbm,
          sidx, didx, buf0, buf1, acc_sh, sem0, sem1):
        c = lax.axis_index("c")
        s = lax.axis_index("s")
        zero16 = jnp.zeros((16,), jnp.float32)

        def run_edges(base, count):
            for p in range(-(-count // _PASS)):
                cnt = min(_PASS, count - p * _PASS)
                rows = pl.ds(base + p * _PASS, cnt)
                pltpu.sync_copy(src_hbm.at[rows], sidx.at[pl.ds(0, cnt)])
                pltpu.sync_copy(dst_hbm.at[rows], didx.at[pl.ds(0, cnt)])
                pltpu.async_copy(h_hbm.at[sidx.at[0]], buf0, sem0)
                pltpu.async_copy(h_hbm.at[sidx.at[1]], buf1, sem1)

                def body(i, _):
                    j = 2 * i
                    for b, buf, sem in ((0, buf0, sem0), (1, buf1, sem1)):
                        e = j + b
                        pltpu.make_async_copy(
                            h_hbm.at[sidx.at[e]], buf, sem).wait()
                        pltpu.sync_copy(buf, acc_sh.at[didx.at[e]], add=True)
                        nk = e + 2

                        @pl.when(nk < cnt)
                        def _():
                            pltpu.async_copy(h_hbm.at[sidx.at[nk]], buf, sem)
                    return 0
                lax.fori_loop(0, cnt // 2, body, 0)

        @pl.when(c == 0)
        def _():
            def zrow(r, _):
                for g in range(8):
                    buf0[r, pl.ds(g * 16, 16)] = zero16
                return 0
            lax.fori_loop(0, 128, zrow, 0)
            for k2 in range(_RCHUNKS):
                pltpu.sync_copy(
                    buf0, acc_sh.at[pl.ds(s * _ROWS_PER_TILE + k2 * 128, 128)])
        plsc.subcore_barrier()

        @pl.when(c == 0)
        def _():
            run_edges(s * split[0], split[0])
        plsc.subcore_barrier()

        @pl.when(c == 0)
        def _():
            for k2 in range(_RCHUNKS):
                rows = pl.ds(s * _ROWS_PER_TILE + k2 * 128, 128)
                pltpu.sync_copy(acc_sh.at[rows], buf0)
                pltpu.sync_copy(buf0, out_hbm.at[rows])

    return k(h, src_a, dst_a)


_BM = 1024


def _row_spec():
    return pl.BlockSpec((_BM, _D), lambda i: (i, 0))


def _tc_prep(hist_e, hist_m):
    def body(he_ref, hm_ref, oe_ref, om_ref):
        oe_ref[...] = lax.rsqrt(jnp.sum(he_ref[...], axis=0) + 1.0)
        om_ref[...] = lax.rsqrt(jnp.sum(hm_ref[...], axis=0) + 1.0)
    return pl.pallas_call(
        body,
        out_shape=[jax.ShapeDtypeStruct((_SHEET, 128), jnp.float32),
                   jax.ShapeDtypeStruct((_SHEET, 128), jnp.float32)],
    )(hist_e, hist_m)


def _tc_in(x, dnb, W):
    def body(x_ref, dn_ref, w_ref, o_ref):
        o_ref[...] = lax.dot_general(
            x_ref[...] * dn_ref[...], w_ref[...],
            (((1,), (0,)), ((), ())), preferred_element_type=jnp.float32)
    return pl.pallas_call(
        body,
        grid=(_NPAD // _BM,),
        in_specs=[_row_spec(), _row_spec(),
                  pl.BlockSpec((_D, _D), lambda i: (0, 0))],
        out_specs=_row_spec(),
        out_shape=jax.ShapeDtypeStruct((_NPAD, _D), jnp.float32),
    )(x, dnb, W)


def _tc_mid(acc, h, dpb, dnb, brow, W, relu):
    def body(a_ref, h_ref, dp_ref, dn_ref, b_ref, w_ref, o_ref):
        pre = dp_ref[...] * (a_ref[...] + h_ref[...]) + b_ref[0:1, :]
        if relu:
            pre = jnp.maximum(pre, 0.0)
        o_ref[...] = lax.dot_general(
            pre * dn_ref[...], w_ref[...],
            (((1,), (0,)), ((), ())), preferred_element_type=jnp.float32)
    return pl.pallas_call(
        body,
        grid=(_NPAD // _BM,),
        in_specs=[_row_spec(), _row_spec(), _row_spec(), _row_spec(),
                  pl.BlockSpec((8, _D), lambda i: (0, 0)),
                  pl.BlockSpec((_D, _D), lambda i: (0, 0))],
        out_specs=_row_spec(),
        out_shape=jax.ShapeDtypeStruct((_NPAD, _D), jnp.float32),
    )(acc, h, dpb, dnb, brow, W)


def _tc_final(acc, h, dpb, brow):
    def body(a_ref, h_ref, dp_ref, b_ref, o_ref):
        z = dp_ref[...] * (a_ref[...] + h_ref[...]) + b_ref[0:1, :]
        m = jnp.max(z, axis=1, keepdims=True)
        e = jnp.exp(z - m)
        ssum = jnp.sum(e, axis=1, keepdims=True)
        o_ref[...] = z - m - jnp.log(ssum)
    return pl.pallas_call(
        body,
        grid=(_NPAD // _BM,),
        in_specs=[_row_spec(), _row_spec(), _row_spec(),
                  pl.BlockSpec((8, _D), lambda i: (0, 0))],
        out_specs=_row_spec(),
        out_shape=jax.ShapeDtypeStruct((_NPAD, _D), jnp.float32),
    )(acc, h, dpb, brow)


def _prep_edges(ei, ch):
    e0 = ei.shape[1]
    totch = _NT * ch
    epad = totch * 128
    src = jnp.concatenate(
        [ei[0], jnp.zeros((epad - e0,), jnp.int32)])
    dst = jnp.concatenate(
        [ei[1], jnp.full((epad - e0,), _N, jnp.int32)])
    return src.reshape(totch, 128), dst.reshape(totch, 128)


def kernel(x, edge_index, meta_edge_index, W1, b1, W2, b2):
    src_e, dst_e = _prep_edges(edge_index, _CH_E)
    src_m, dst_m = _prep_edges(meta_edge_index, _CH_M)
    xp = jnp.pad(x, ((0, _NPAD - _N), (0, 0)))
    b1r = jnp.broadcast_to(b1[None, :], (8, _D))
    b2r = jnp.broadcast_to(b2[None, :], (8, _D))

    hist_e, hist_m = _sc_hist(dst_e, dst_m)
    dis_e, dis_m = _tc_prep(hist_e, hist_m)
    de_b = jnp.broadcast_to(dis_e.reshape(-1)[:, None], (_NPAD, _D))
    dm_b = jnp.broadcast_to(dis_m.reshape(-1)[:, None], (_NPAD, _D))

    h1 = _tc_in(xp, de_b, W1)
    a1 = _sc_scatter(h1, src_e, dst_e, _SPLIT_E)
    h2 = _tc_mid(a1, h1, de_b, de_b, b1r, W2, relu=True)
    a2 = _sc_scatter(h2, src_e, dst_e, _SPLIT_E)
    h3 = _tc_mid(a2, h2, de_b, dm_b, b2r, W1, relu=False)
    a3 = _sc_scatter(h3, src_m, dst_m, _SPLIT_M)
    h4 = _tc_mid(a3, h3, dm_b, dm_b, b1r, W2, relu=True)
    a4 = _sc_scatter(h4, src_m, dst_m, _SPLIT_M)
    out = _tc_final(a4, h4, dm_b, b2r)
    return out[:_N]


# speedup vs baseline: 3.2061x; 3.2061x over previous
import functools

import jax
import jax.numpy as jnp
from jax import lax
from jax.experimental import pallas as pl
from jax.experimental.pallas import tpu as pltpu
from jax.experimental.pallas import tpu_sc as plsc

_N = 10000
_D = 128
_NPAD = 10240
_SHEET = _NPAD // 128
_NC = 2
_NS = 16
_NT = _NC * _NS
_ROWS_PER_TILE = _NPAD // _NS
_RCHUNKS = _ROWS_PER_TILE // 128

_CH_E = 80
_CH_M = 40
_PASS = 40
_SPLIT_E = (80, 80)
_SPLIT_M = (40, 40)


def _mesh():
    return plsc.VectorSubcoreMesh(
        core_axis_name="c", subcore_axis_name="s",
        num_cores=_NC, num_subcores=_NS)


def _sc_hist(dst_e, dst_m):
    chmax = max(_CH_E, _CH_M)

    @functools.partial(
        pl.kernel,
        out_type=[jax.ShapeDtypeStruct((_NT, _SHEET, 128), jnp.float32),
                  jax.ShapeDtypeStruct((_NT, _SHEET, 128), jnp.float32)],
        mesh=_mesh(),
        compiler_params=pltpu.CompilerParams(needs_layout_passes=False),
        name="sc_hist",
        scratch_types=[
            pltpu.VMEM((chmax, 128), jnp.int32),
            pltpu.VMEM((_NPAD,), jnp.float32),
            pltpu.VMEM((_SHEET, 128), jnp.float32),
        ],
    )
    def k(dst_e_hbm, dst_m_hbm, out_e_hbm, out_m_hbm, dstv, histl, hist2d):
        c = lax.axis_index("c")
        s = lax.axis_index("s")
        wid = s * _NC + c
        zero16 = jnp.zeros((16,), jnp.float32)
        ones16 = jnp.ones((16,), jnp.float32)

        def run_graph(dst_hbm, ch, out_hbm):
            def zrow(r, _):
                for g in range(8):
                    histl[pl.ds(r * 128 + g * 16, 16)] = zero16
                return 0
            lax.fori_loop(0, _SHEET, zrow, 0)
            pltpu.sync_copy(dst_hbm.at[pl.ds(wid * ch, ch)],
                            dstv.at[pl.ds(0, ch)])

            def erow(r, _):
                for g in range(8):
                    d = dstv[r, pl.ds(g * 16, 16)]
                    plsc.addupdate_scatter(histl, [d], ones16)
                return 0
            lax.fori_loop(0, ch, erow, 0)

            def prow(r, _):
                for g in range(8):
                    hist2d[r, pl.ds(g * 16, 16)] = (
                        histl[pl.ds(r * 128 + g * 16, 16)])
                return 0
            lax.fori_loop(0, _SHEET, prow, 0)
            pltpu.sync_copy(hist2d, out_hbm.at[wid])

        run_graph(dst_e_hbm, _CH_E, out_e_hbm)
        run_graph(dst_m_hbm, _CH_M, out_m_hbm)

    return k(dst_e, dst_m)


def _sc_scatter(h, src_a, dst_a, split):
    @functools.partial(
        pl.kernel,
        out_type=jax.ShapeDtypeStruct((_NC, _NPAD, 128), jnp.float32),
        mesh=_mesh(),
        scratch_types=[
            pltpu.VMEM((_PASS, 128), jnp.int32),
            pltpu.VMEM((_PASS, 128), jnp.int32),
            pltpu.VMEM((128, 128), jnp.float32),
            pltpu.VMEM((128, 128), jnp.float32),
            pltpu.VMEM_SHARED((_NPAD, 128), jnp.float32),
            pltpu.SemaphoreType.DMA,
            pltpu.SemaphoreType.DMA,
        ],
    )
    def k(h_hbm, src_hbm, dst_hbm, out_hbm,
          sidx, didx, buf0, buf1, acc_sh, sem0, sem1):
        c = lax.axis_index("c")
        s = lax.axis_index("s")
        zero16 = jnp.zeros((16,), jnp.float32)

        def run_edges(base, count):
            for p in range(-(-count // _PASS)):
                cnt = min(_PASS, count - p * _PASS)
                rows = pl.ds(base + p * _PASS, cnt)
                pltpu.sync_copy(src_hbm.at[rows], sidx.at[pl.ds(0, cnt)])
                pltpu.sync_copy(dst_hbm.at[rows], didx.at[pl.ds(0, cnt)])
                pltpu.async_copy(h_hbm.at[sidx.at[0]], buf0, sem0)
                pltpu.async_copy(h_hbm.at[sidx.at[1]], buf1, sem1)

                def body(i, _):
                    j = 2 * i
                    for b, buf, sem in ((0, buf0, sem0), (1, buf1, sem1)):
                        e = j + b
                        pltpu.make_async_copy(
                            h_hbm.at[sidx.at[e]], buf, sem).wait()
                        pltpu.sync_copy(buf, acc_sh.at[didx.at[e]], add=True)
                        nk = e + 2

                        @pl.when(nk < cnt)
                        def _():
                            pltpu.async_copy(h_hbm.at[sidx.at[nk]], buf, sem)
                    return 0
                lax.fori_loop(0, cnt // 2, body, 0)

        def zrow(r, _):
            for g in range(8):
                buf0[r, pl.ds(g * 16, 16)] = zero16
            return 0
        lax.fori_loop(0, 128, zrow, 0)
        for k2 in range(_RCHUNKS):
            pltpu.sync_copy(
                buf0, acc_sh.at[pl.ds(s * _ROWS_PER_TILE + k2 * 128, 128)])
        plsc.subcore_barrier()

        @pl.when(c == 0)
        def _():
            run_edges(s * split[0], split[0])

        @pl.when(c == 1)
        def _():
            run_edges(_NS * split[0] + s * split[1], split[1])
        plsc.subcore_barrier()

        for k2 in range(_RCHUNKS):
            rows = pl.ds(s * _ROWS_PER_TILE + k2 * 128, 128)
            pltpu.sync_copy(acc_sh.at[rows], buf0)
            pltpu.sync_copy(buf0, out_hbm.at[c, rows])

    return k(h, src_a, dst_a)


_BM = 1024


def _row_spec():
    return pl.BlockSpec((_BM, _D), lambda i: (i, 0))


def _tc_prep(hist_e, hist_m):
    def body(he_ref, hm_ref, oe_ref, om_ref):
        oe_ref[...] = lax.rsqrt(jnp.sum(he_ref[...], axis=0) + 1.0)
        om_ref[...] = lax.rsqrt(jnp.sum(hm_ref[...], axis=0) + 1.0)
    return pl.pallas_call(
        body,
        out_shape=[jax.ShapeDtypeStruct((_SHEET, 128), jnp.float32),
                   jax.ShapeDtypeStruct((_SHEET, 128), jnp.float32)],
    )(hist_e, hist_m)


def _tc_in(x, dnb, W):
    def body(x_ref, dn_ref, w_ref, o_ref):
        o_ref[...] = lax.dot_general(
            x_ref[...] * dn_ref[...], w_ref[...],
            (((1,), (0,)), ((), ())), preferred_element_type=jnp.float32)
    return pl.pallas_call(
        body,
        grid=(_NPAD // _BM,),
        in_specs=[_row_spec(), _row_spec(),
                  pl.BlockSpec((_D, _D), lambda i: (0, 0))],
        out_specs=_row_spec(),
        out_shape=jax.ShapeDtypeStruct((_NPAD, _D), jnp.float32),
    )(x, dnb, W)


def _tc_mid(acc0, acc1, h, dpb, dnb, brow, W, relu):
    def body(a0_ref, a1_ref, h_ref, dp_ref, dn_ref, b_ref, w_ref, o_ref):
        pre = dp_ref[...] * (a0_ref[...] + a1_ref[...] + h_ref[...]) + b_ref[0:1, :]
        if relu:
            pre = jnp.maximum(pre, 0.0)
        o_ref[...] = lax.dot_general(
            pre * dn_ref[...], w_ref[...],
            (((1,), (0,)), ((), ())), preferred_element_type=jnp.float32)
    return pl.pallas_call(
        body,
        grid=(_NPAD // _BM,),
        in_specs=[_row_spec(), _row_spec(), _row_spec(), _row_spec(),
                  _row_spec(),
                  pl.BlockSpec((8, _D), lambda i: (0, 0)),
                  pl.BlockSpec((_D, _D), lambda i: (0, 0))],
        out_specs=_row_spec(),
        out_shape=jax.ShapeDtypeStruct((_NPAD, _D), jnp.float32),
    )(acc0, acc1, h, dpb, dnb, brow, W)


def _tc_final(acc0, acc1, h, dpb, brow):
    def body(a0_ref, a1_ref, h_ref, dp_ref, b_ref, o_ref):
        z = dp_ref[...] * (a0_ref[...] + a1_ref[...] + h_ref[...]) + b_ref[0:1, :]
        m = jnp.max(z, axis=1, keepdims=True)
        e = jnp.exp(z - m)
        ssum = jnp.sum(e, axis=1, keepdims=True)
        o_ref[...] = z - m - jnp.log(ssum)
    return pl.pallas_call(
        body,
        grid=(_NPAD // _BM,),
        in_specs=[_row_spec(), _row_spec(), _row_spec(), _row_spec(),
                  pl.BlockSpec((8, _D), lambda i: (0, 0))],
        out_specs=_row_spec(),
        out_shape=jax.ShapeDtypeStruct((_NPAD, _D), jnp.float32),
    )(acc0, acc1, h, dpb, brow)


def _prep_edges(ei, ch):
    e0 = ei.shape[1]
    totch = _NT * ch
    epad = totch * 128
    npad_edges = epad - e0
    pad_ids = jnp.arange(npad_edges, dtype=jnp.int32)
    src = jnp.concatenate([ei[0], pad_ids % _N])
    dst = jnp.concatenate([ei[1], _N + pad_ids % (_NPAD - _N)])
    return src.reshape(totch, 128), dst.reshape(totch, 128)


def kernel(x, edge_index, meta_edge_index, W1, b1, W2, b2):
    src_e, dst_e = _prep_edges(edge_index, _CH_E)
    src_m, dst_m = _prep_edges(meta_edge_index, _CH_M)
    xp = jnp.pad(x, ((0, _NPAD - _N), (0, 0)))
    b1r = jnp.broadcast_to(b1[None, :], (8, _D))
    b2r = jnp.broadcast_to(b2[None, :], (8, _D))

    hist_e, hist_m = _sc_hist(dst_e, dst_m)
    dis_e, dis_m = _tc_prep(hist_e, hist_m)
    de_b = jnp.broadcast_to(dis_e.reshape(-1)[:, None], (_NPAD, _D))
    dm_b = jnp.broadcast_to(dis_m.reshape(-1)[:, None], (_NPAD, _D))

    h1 = _tc_in(xp, de_b, W1)
    a1 = _sc_scatter(h1, src_e, dst_e, _SPLIT_E)
    h2 = _tc_mid(a1[0], a1[1], h1, de_b, de_b, b1r, W2, relu=True)
    a2 = _sc_scatter(h2, src_e, dst_e, _SPLIT_E)
    h3 = _tc_mid(a2[0], a2[1], h2, de_b, dm_b, b2r, W1, relu=False)
    a3 = _sc_scatter(h3, src_m, dst_m, _SPLIT_M)
    h4 = _tc_mid(a3[0], a3[1], h3, dm_b, dm_b, b1r, W2, relu=True)
    a4 = _sc_scatter(h4, src_m, dst_m, _SPLIT_M)
    out = _tc_final(a4[0], a4[1], h4, dm_b, b2r)
    return out[:_N]


# speedup vs baseline: 3.4490x; 1.0758x over previous
import functools

import jax
import jax.numpy as jnp
from jax import lax
from jax.experimental import pallas as pl
from jax.experimental.pallas import tpu as pltpu
from jax.experimental.pallas import tpu_sc as plsc

_N = 10000
_D = 128
_NPAD = 10240
_SHEET = _NPAD // 128
_NC = 2
_NS = 16
_NT = _NC * _NS
_ROWS_PER_TILE = _NPAD // _NS
_RCHUNKS = _ROWS_PER_TILE // 128

_CH_E = 80
_CH_M = 40
_PASS = 40
_SPLIT_E = (80, 80)
_SPLIT_M = (40, 40)


def _mesh():
    return plsc.VectorSubcoreMesh(
        core_axis_name="c", subcore_axis_name="s",
        num_cores=_NC, num_subcores=_NS)


def _sc_hist(dst_e, dst_m):
    chmax = max(_CH_E, _CH_M)

    @functools.partial(
        pl.kernel,
        out_type=[jax.ShapeDtypeStruct((_NT, _SHEET, 128), jnp.float32),
                  jax.ShapeDtypeStruct((_NT, _SHEET, 128), jnp.float32)],
        mesh=_mesh(),
        compiler_params=pltpu.CompilerParams(needs_layout_passes=False),
        name="sc_hist",
        scratch_types=[
            pltpu.VMEM((chmax, 128), jnp.int32),
            pltpu.VMEM((_NPAD,), jnp.float32),
            pltpu.VMEM((_SHEET, 128), jnp.float32),
        ],
    )
    def k(dst_e_hbm, dst_m_hbm, out_e_hbm, out_m_hbm, dstv, histl, hist2d):
        c = lax.axis_index("c")
        s = lax.axis_index("s")
        wid = s * _NC + c
        zero16 = jnp.zeros((16,), jnp.float32)
        ones16 = jnp.ones((16,), jnp.float32)

        def run_graph(dst_hbm, ch, out_hbm):
            def zrow(r, _):
                for g in range(8):
                    histl[pl.ds(r * 128 + g * 16, 16)] = zero16
                return 0
            lax.fori_loop(0, _SHEET, zrow, 0)
            pltpu.sync_copy(dst_hbm.at[pl.ds(wid * ch, ch)],
                            dstv.at[pl.ds(0, ch)])

            def erow(r, _):
                for g in range(8):
                    d = dstv[r, pl.ds(g * 16, 16)]
                    plsc.addupdate_scatter(histl, [d], ones16)
                return 0
            lax.fori_loop(0, ch, erow, 0)

            def prow(r, _):
                for g in range(8):
                    hist2d[r, pl.ds(g * 16, 16)] = (
                        histl[pl.ds(r * 128 + g * 16, 16)])
                return 0
            lax.fori_loop(0, _SHEET, prow, 0)
            pltpu.sync_copy(hist2d, out_hbm.at[wid])

        run_graph(dst_e_hbm, _CH_E, out_e_hbm)
        run_graph(dst_m_hbm, _CH_M, out_m_hbm)

    return k(dst_e, dst_m)


def _sc_scatter(h, src_a, dst_a, split):
    @functools.partial(
        pl.kernel,
        out_type=jax.ShapeDtypeStruct((_NC, _NPAD, 128), jnp.float32),
        mesh=_mesh(),
        scratch_types=[
            pltpu.VMEM((_PASS, 128), jnp.int32),
            pltpu.VMEM((_PASS, 128), jnp.int32),
            pltpu.VMEM((128, 128), jnp.float32),
            pltpu.VMEM((128, 128), jnp.float32),
            pltpu.VMEM_SHARED((_NPAD, 128), jnp.float32),
            pltpu.SemaphoreType.DMA,
            pltpu.SemaphoreType.DMA,
        ],
    )
    def k(h_hbm, src_hbm, dst_hbm, out_hbm,
          sidx, didx, buf0, buf1, acc_sh, sem0, sem1):
        c = lax.axis_index("c")
        s = lax.axis_index("s")
        zero16 = jnp.zeros((16,), jnp.float32)

        def run_edges(base, count):
            for p in range(-(-count // _PASS)):
                cnt = min(_PASS, count - p * _PASS)
                rows = pl.ds(base + p * _PASS, cnt)
                pltpu.sync_copy(src_hbm.at[rows], sidx.at[pl.ds(0, cnt)])
                pltpu.sync_copy(dst_hbm.at[rows], didx.at[pl.ds(0, cnt)])
                pltpu.async_copy(h_hbm.at[sidx.at[0]], buf0, sem0)
                pltpu.async_copy(h_hbm.at[sidx.at[1]], buf1, sem1)

                def body(i, _):
                    j = 2 * i
                    for b, buf, sem in ((0, buf0, sem0), (1, buf1, sem1)):
                        e = j + b
                        pltpu.make_async_copy(
                            h_hbm.at[sidx.at[e]], buf, sem).wait()
                        pltpu.sync_copy(buf, acc_sh.at[didx.at[e]], add=True)
                        nk = e + 2

                        @pl.when(nk < cnt)
                        def _():
                            pltpu.async_copy(h_hbm.at[sidx.at[nk]], buf, sem)
                    return 0
                lax.fori_loop(0, cnt // 2, body, 0)

        def zrow(r, _):
            for g in range(8):
                buf0[r, pl.ds(g * 16, 16)] = zero16
            return 0
        lax.fori_loop(0, 128, zrow, 0)
        for k2 in range(_RCHUNKS):
            pltpu.sync_copy(
                buf0, acc_sh.at[pl.ds(s * _ROWS_PER_TILE + k2 * 128, 128)])
        plsc.subcore_barrier()

        @pl.when(c == 0)
        def _():
            run_edges(s * split[0], split[0])

        @pl.when(c == 1)
        def _():
            run_edges(_NS * split[0] + s * split[1], split[1])
        plsc.subcore_barrier()

        for k2 in range(_RCHUNKS):
            rows = pl.ds(s * _ROWS_PER_TILE + k2 * 128, 128)
            pltpu.sync_copy(acc_sh.at[rows], buf0)
            pltpu.sync_copy(buf0, out_hbm.at[c, rows])

    return k(h, src_a, dst_a)


_BM = 1024


def _row_spec():
    return pl.BlockSpec((_BM, _D), lambda i: (i, 0))


def _tc_prep(hist_e, hist_m):
    def body(he_ref, hm_ref, oe_ref, om_ref):
        oe_ref[...] = lax.rsqrt(jnp.sum(he_ref[...], axis=0) + 1.0)
        om_ref[...] = lax.rsqrt(jnp.sum(hm_ref[...], axis=0) + 1.0)
    return pl.pallas_call(
        body,
        out_shape=[jax.ShapeDtypeStruct((_SHEET, 128), jnp.float32),
                   jax.ShapeDtypeStruct((_SHEET, 128), jnp.float32)],
    )(hist_e, hist_m)


def _tc_in(x, dnc, W):
    def body(x_ref, dn_ref, w_ref, o_ref):
        o_ref[...] = lax.dot_general(
            x_ref[...] * dn_ref[...], w_ref[...],
            (((1,), (0,)), ((), ())), preferred_element_type=jnp.float32)
    return pl.pallas_call(
        body,
        grid=(_NPAD // _BM,),
        in_specs=[_row_spec(), _col_spec(),
                  pl.BlockSpec((_D, _D), lambda i: (0, 0))],
        out_specs=_row_spec(),
        out_shape=jax.ShapeDtypeStruct((_NPAD, _D), jnp.float32),
    )(x, dnc, W)


def _acc_spec(part):
    return pl.BlockSpec((1, _BM, _D), lambda i, p=part: (p, i, 0))


def _col_spec():
    return pl.BlockSpec((_BM, 1), lambda i: (i, 0))


def _tc_mid(accs, h, dpc, dnc, brow, W, relu):
    def body(a0_ref, a1_ref, h_ref, dp_ref, dn_ref, b_ref, w_ref, o_ref):
        acc = a0_ref[0] + a1_ref[0]
        pre = dp_ref[...] * (acc + h_ref[...]) + b_ref[0:1, :]
        if relu:
            pre = jnp.maximum(pre, 0.0)
        o_ref[...] = lax.dot_general(
            pre * dn_ref[...], w_ref[...],
            (((1,), (0,)), ((), ())), preferred_element_type=jnp.float32)
    return pl.pallas_call(
        body,
        grid=(_NPAD // _BM,),
        in_specs=[_acc_spec(0), _acc_spec(1), _row_spec(), _col_spec(),
                  _col_spec(),
                  pl.BlockSpec((8, _D), lambda i: (0, 0)),
                  pl.BlockSpec((_D, _D), lambda i: (0, 0))],
        out_specs=_row_spec(),
        out_shape=jax.ShapeDtypeStruct((_NPAD, _D), jnp.float32),
    )(accs, accs, h, dpc, dnc, brow, W)


def _tc_final(accs, h, dpc, brow):
    def body(a0_ref, a1_ref, h_ref, dp_ref, b_ref, o_ref):
        acc = a0_ref[0] + a1_ref[0]
        z = dp_ref[...] * (acc + h_ref[...]) + b_ref[0:1, :]
        m = jnp.max(z, axis=1, keepdims=True)
        e = jnp.exp(z - m)
        ssum = jnp.sum(e, axis=1, keepdims=True)
        o_ref[...] = z - m - jnp.log(ssum)
    return pl.pallas_call(
        body,
        grid=(_NPAD // _BM,),
        in_specs=[_acc_spec(0), _acc_spec(1), _row_spec(), _col_spec(),
                  pl.BlockSpec((8, _D), lambda i: (0, 0))],
        out_specs=_row_spec(),
        out_shape=jax.ShapeDtypeStruct((_N, _D), jnp.float32),
    )(accs, accs, h, dpc, brow)


def _prep_edges(ei, ch):
    e0 = ei.shape[1]
    totch = _NT * ch
    epad = totch * 128
    npad_edges = epad - e0
    pad_ids = jnp.arange(npad_edges, dtype=jnp.int32)
    src = jnp.concatenate([ei[0], pad_ids % _N])
    dst = jnp.concatenate([ei[1], _N + pad_ids % (_NPAD - _N)])
    return src.reshape(totch, 128), dst.reshape(totch, 128)


def kernel(x, edge_index, meta_edge_index, W1, b1, W2, b2):
    src_e, dst_e = _prep_edges(edge_index, _CH_E)
    src_m, dst_m = _prep_edges(meta_edge_index, _CH_M)
    xp = jnp.pad(x, ((0, _NPAD - _N), (0, 0)))
    b1r = jnp.broadcast_to(b1[None, :], (8, _D))
    b2r = jnp.broadcast_to(b2[None, :], (8, _D))

    hist_e, hist_m = _sc_hist(dst_e, dst_m)
    dis_e, dis_m = _tc_prep(hist_e, hist_m)
    de_c = dis_e.reshape(_NPAD, 1)
    dm_c = dis_m.reshape(_NPAD, 1)

    h1 = _tc_in(xp, de_c, W1)
    a1 = _sc_scatter(h1, src_e, dst_e, _SPLIT_E)
    h2 = _tc_mid(a1, h1, de_c, de_c, b1r, W2, relu=True)
    a2 = _sc_scatter(h2, src_e, dst_e, _SPLIT_E)
    h3 = _tc_mid(a2, h2, de_c, dm_c, b2r, W1, relu=False)
    a3 = _sc_scatter(h3, src_m, dst_m, _SPLIT_M)
    h4 = _tc_mid(a3, h3, dm_c, dm_c, b1r, W2, relu=True)
    a4 = _sc_scatter(h4, src_m, dst_m, _SPLIT_M)
    return _tc_final(a4, h4, dm_c, b2r)
